# Initial kernel scaffold; baseline (speedup 1.0000x reference)
import jax
import jax.numpy as jnp
from jax.experimental import pallas as pl


def kernel(x, edge_index, edge_attr, W_l1, b_l1, W_r1, W_e1, att1, bias1, W_l2, b_l2, W_r2, W_e2, att2, bias2, Wm1, bm1, Wm2, bm2):
    raise NotImplementedError("write your pallas kernel here")


# speedup vs baseline: 5.3498x; 5.3498x over previous
import functools

import jax
import jax.numpy as jnp
from jax import lax
from jax.experimental import pallas as pl
from jax.experimental.pallas import tpu as pltpu
from jax.experimental.pallas import tpu_sc as plsc

N = 10000
E = 320000
D = 128
ED = 16

NC = 2
NS = 16
NW = NC * NS
EW = E // NW
CH = 80
NCHUNK = EW // CH
NP = 10240
NPR = NP // 128
NSTRIPE = NP // NS
SCH = 80
NSCH = NSTRIPE // SCH
DSTRIPE = 8

_mesh = plsc.VectorSubcoreMesh(core_axis_name="c", subcore_axis_name="s",
                               num_cores=NC, num_subcores=NS)


def _dense_nodes_body(x_ref, wl_ref, bl_ref, wr_ref, ul_ref, ur_ref):
    x = x_ref[...]
    ul_ref[...] = jnp.dot(x, wl_ref[...], preferred_element_type=jnp.float32) + bl_ref[...]
    ur_ref[...] = jnp.dot(x, wr_ref[...], preferred_element_type=jnp.float32)


def _dense_nodes(x, wl, bl, wr, nb):
    grid = N // nb
    return pl.pallas_call(
        _dense_nodes_body,
        grid=(grid,),
        in_specs=[
            pl.BlockSpec((nb, D), lambda i: (i, 0)),
            pl.BlockSpec((D, D), lambda i: (0, 0)),
            pl.BlockSpec((1, D), lambda i: (0, 0)),
            pl.BlockSpec((D, D), lambda i: (0, 0)),
        ],
        out_specs=[
            pl.BlockSpec((nb, D), lambda i: (i, 0)),
            pl.BlockSpec((nb, D), lambda i: (i, 0)),
        ],
        out_shape=[
            jax.ShapeDtypeStruct((N, D), jnp.float32),
            jax.ShapeDtypeStruct((N, D), jnp.float32),
        ],
    )(x, wl, bl.reshape(1, D), wr)


def _edge_stage_body(gl_ref, gr_ref, ea_ref, we_ref, att_ref, x_ref, wd_ref):
    gl = gl_ref[...]
    ue = jnp.dot(ea_ref[...], we_ref[...], preferred_element_type=jnp.float32)
    m = gl + gr_ref[...] + ue
    m = jnp.where(m >= 0.0, m, 0.2 * m)
    alpha = jnp.sum(m * att_ref[...], axis=1, keepdims=True)
    w = jnp.exp(alpha)
    x_ref[...] = w * gl
    wd_ref[...] = w


def _edge_stage(gl, gr, ea, we, att, eb):
    grid = E // eb
    return pl.pallas_call(
        _edge_stage_body,
        grid=(grid,),
        in_specs=[
            pl.BlockSpec((eb, D), lambda i: (i, 0)),
            pl.BlockSpec((eb, D), lambda i: (i, 0)),
            pl.BlockSpec((eb, ED), lambda i: (i, 0)),
            pl.BlockSpec((ED, D), lambda i: (0, 0)),
            pl.BlockSpec((1, D), lambda i: (0, 0)),
        ],
        out_specs=[
            pl.BlockSpec((eb, D), lambda i: (i, 0)),
            pl.BlockSpec((eb, 1), lambda i: (i, 0)),
        ],
        out_shape=[
            jax.ShapeDtypeStruct((E, D), jnp.float32),
            jax.ShapeDtypeStruct((E, 1), jnp.float32),
        ],
    )(gl, gr, ea, we, att.reshape(1, D))


def _node_update_body(elu, sn_ref, sd_ref, bias_ref, wl_ref, bl_ref, wr_ref,
                      ul_ref, ur_ref):
    num = sn_ref[0] + sn_ref[1]
    den = sd_ref[0] + sd_ref[1]
    h = num / (den + 1e-16) + bias_ref[...]
    if elu:
        h = jnp.where(h > 0.0, h, jnp.exp(h) - 1.0)
    ul_ref[...] = jnp.dot(h, wl_ref[...], preferred_element_type=jnp.float32) + bl_ref[...]
    ur_ref[...] = jnp.dot(h, wr_ref[...], preferred_element_type=jnp.float32)


def _node_update(sn, sd, bias, wl, bl, wr, nb, elu):
    grid = NP // nb
    return pl.pallas_call(
        functools.partial(_node_update_body, elu),
        grid=(grid,),
        in_specs=[
            pl.BlockSpec((NC, nb, D), lambda i: (0, i, 0)),
            pl.BlockSpec((NC, nb, 1), lambda i: (0, i, 0)),
            pl.BlockSpec((1, D), lambda i: (0, 0)),
            pl.BlockSpec((D, D), lambda i: (0, 0)),
            pl.BlockSpec((1, D), lambda i: (0, 0)),
            pl.BlockSpec((D, D), lambda i: (0, 0)),
        ],
        out_specs=[
            pl.BlockSpec((nb, D), lambda i: (i, 0)),
            pl.BlockSpec((nb, D), lambda i: (i, 0)),
        ],
        out_shape=[
            jax.ShapeDtypeStruct((NP, D), jnp.float32),
            jax.ShapeDtypeStruct((NP, D), jnp.float32),
        ],
    )(sn, sd, bias.reshape(1, D), wl, bl.reshape(1, D), wr)


def _pred_stage_body(gp_ref, gq_ref, wm2_ref, bm2_ref, out_ref):
    s = jnp.maximum(gp_ref[...] + gq_ref[...], 0.0)
    out_ref[...] = jnp.sum(s * wm2_ref[...], axis=1, keepdims=True) + bm2_ref[...]


def _pred_stage(gp, gq, wm2, bm2, eb):
    grid = E // eb
    return pl.pallas_call(
        _pred_stage_body,
        grid=(grid,),
        in_specs=[
            pl.BlockSpec((eb, D), lambda i: (i, 0)),
            pl.BlockSpec((eb, D), lambda i: (i, 0)),
            pl.BlockSpec((1, D), lambda i: (0, 0)),
            pl.BlockSpec((1, 1), lambda i: (0, 0)),
        ],
        out_specs=pl.BlockSpec((eb, 1), lambda i: (i, 0)),
        out_shape=jax.ShapeDtypeStruct((E, 1), jnp.float32),
    )(gp, gq, wm2.reshape(1, D), bm2.reshape(1, 1))


@functools.partial(
    pl.kernel,
    out_type=[
        jax.ShapeDtypeStruct((E, D), jnp.float32),
        jax.ShapeDtypeStruct((E, D), jnp.float32),
    ],
    mesh=_mesh,
    scratch_types=[
        pltpu.VMEM((NCHUNK, CH), jnp.int32),
        pltpu.VMEM((NCHUNK, CH), jnp.int32),
        pltpu.VMEM((CH, D), jnp.float32),
        pltpu.SemaphoreType.DMA,
    ],
)
def _sc_gather2(taba_hbm, tabb_hbm, idxa_hbm, idxb_hbm, outa_hbm, outb_hbm,
                idxa_v, idxb_v, rows_v, sem):
    wid = lax.axis_index("s") * NC + lax.axis_index("c")
    pltpu.sync_copy(idxa_hbm.at[wid], idxa_v)
    pltpu.sync_copy(idxb_hbm.at[wid], idxb_v)
    base = wid * EW

    def body(j, carry):
        pltpu.async_copy(taba_hbm.at[idxa_v.at[j]], rows_v, sem).wait()
        pltpu.sync_copy(rows_v, outa_hbm.at[pl.ds(base + j * CH, CH)])
        pltpu.async_copy(tabb_hbm.at[idxb_v.at[j]], rows_v, sem).wait()
        pltpu.sync_copy(rows_v, outb_hbm.at[pl.ds(base + j * CH, CH)])
        return carry

    lax.fori_loop(0, NCHUNK, body, 0)


@functools.partial(
    pl.kernel,
    out_type=[
        jax.ShapeDtypeStruct((NC, NP, D), jnp.float32),
        jax.ShapeDtypeStruct((NC, NPR, 128), jnp.float32),
    ],
    mesh=_mesh,
    compiler_params=pltpu.CompilerParams(needs_layout_passes=False),
    scratch_types=[
        pltpu.VMEM_SHARED((NP, D), jnp.float32),
        pltpu.VMEM_SHARED((NPR, 128), jnp.float32),
        pltpu.VMEM((CH,), jnp.int32),
        pltpu.VMEM((CH, D), jnp.float32),
        pltpu.VMEM((CH,), jnp.float32),
        pltpu.VMEM((NPR, 128), jnp.float32),
        pltpu.VMEM((NPR,), jnp.int32),
    ],
)
def _sc_scatter(x_hbm, w_hbm, idx_hbm, zn_hbm, iota_hbm, sn_hbm, sd_hbm,
                accn_sh, accd_sh, idxc_v, ebuf_v, wbuf_v, den_v, iota_v):
    cid = lax.axis_index("c")
    sid = lax.axis_index("s")
    wid = sid * NC + cid
    stripe = sid * NSTRIPE
    base = wid * EW

    pltpu.sync_copy(zn_hbm, ebuf_v)
    pltpu.sync_copy(zn_hbm.at[pl.ds(0, NPR)], den_v)
    pltpu.sync_copy(iota_hbm, iota_v)

    def zbody(t, carry):
        pltpu.sync_copy(ebuf_v, accn_sh.at[pl.ds(stripe + t * SCH, SCH)])
        return carry

    lax.fori_loop(0, NSCH, zbody, 0)

    @pl.when(sid < NPR // DSTRIPE)
    def _():
        pltpu.sync_copy(ebuf_v.at[pl.ds(0, DSTRIPE)],
                        accd_sh.at[pl.ds(sid * DSTRIPE, DSTRIPE)])

    plsc.subcore_barrier()

    def body(j, carry):
        pltpu.sync_copy(idx_hbm.at[wid].at[j], idxc_v)
        pltpu.sync_copy(x_hbm.at[pl.ds(base + j * CH, CH)], ebuf_v)
        pltpu.sync_copy(ebuf_v, accn_sh.at[idxc_v], add=True)
        pltpu.sync_copy(w_hbm.at[wid].at[j], wbuf_v)
        for g in range(CH // 16):
            idx16 = idxc_v[pl.ds(g * 16, 16)]
            w16 = wbuf_v[pl.ds(g * 16, 16)]
            hi = lax.shift_right_logical(idx16, 7)
            lo = lax.bitwise_and(idx16, 127)
            plsc.addupdate_scatter(den_v, [hi, lo], w16)
        return carry

    lax.fori_loop(0, NCHUNK, body, 0)

    pltpu.sync_copy(den_v, accd_sh.at[iota_v], add=True)
    plsc.subcore_barrier()

    def obody(t, carry):
        pltpu.sync_copy(accn_sh.at[pl.ds(stripe + t * SCH, SCH)], ebuf_v)
        pltpu.sync_copy(ebuf_v, sn_hbm.at[cid].at[pl.ds(stripe + t * SCH, SCH)])
        return carry

    lax.fori_loop(0, NSCH, obody, 0)

    @pl.when(sid < NPR // DSTRIPE)
    def _():
        pltpu.sync_copy(accd_sh.at[pl.ds(sid * DSTRIPE, DSTRIPE)],
                        den_v.at[pl.ds(0, DSTRIPE)])
        pltpu.sync_copy(den_v.at[pl.ds(0, DSTRIPE)],
                        sd_hbm.at[cid].at[pl.ds(sid * DSTRIPE, DSTRIPE)])


def kernel(x, edge_index, edge_attr, W_l1, b_l1, W_r1, W_e1, att1, bias1,
           W_l2, b_l2, W_r2, W_e2, att2, bias2, Wm1, bm1, Wm2, bm2):
    src3 = edge_index[0].reshape(NW, NCHUNK, CH)
    dst3 = edge_index[1].reshape(NW, NCHUNK, CH)
    zn = jnp.zeros((SCH, D), jnp.float32)
    iota = jnp.arange(NPR, dtype=jnp.int32)

    ul1, ur1 = _dense_nodes(x, W_l1, b_l1, W_r1, nb=2000)
    gl1, gr1 = _sc_gather2(ul1, ur1, src3, dst3)
    x1, w1 = _edge_stage(gl1, gr1, edge_attr, W_e1, att1.reshape(D), eb=8000)
    sn1, sd1 = _sc_scatter(x1, w1.reshape(NW, NCHUNK, CH), dst3, zn, iota)

    ul2, ur2 = _node_update(sn1, sd1.reshape(NC, NP, 1), bias1, W_l2, b_l2,
                            W_r2, nb=2048, elu=True)
    gl2, gr2 = _sc_gather2(ul2, ur2, src3, dst3)
    x2, w2 = _edge_stage(gl2, gr2, edge_attr, W_e2, att2.reshape(D), eb=8000)
    sn2, sd2 = _sc_scatter(x2, w2.reshape(NW, NCHUNK, CH), dst3, zn, iota)

    p, q = _node_update(sn2, sd2.reshape(NC, NP, 1), bias2, Wm1[:D], bm1,
                        Wm1[D:], nb=2048, elu=False)
    gp, gq = _sc_gather2(p, q, src3, dst3)
    return _pred_stage(gp, gq, Wm2.reshape(D), bm2, eb=8000)


# speedup vs baseline: 6.0876x; 1.1379x over previous
import functools

import jax
import jax.numpy as jnp
from jax import lax
from jax.experimental import pallas as pl
from jax.experimental.pallas import tpu as pltpu
from jax.experimental.pallas import tpu_sc as plsc

N = 10000
E = 320000
D = 128
ED = 16

NC = 2
NS = 16
NW = NC * NS
EW = E // NW
CH = 80
NCHUNK = EW // CH
NP = 10240
NPR = NP // 128
NSTRIPE = NP // NS
SCH = 80
NSCH = NSTRIPE // SCH
DSTRIPE = 8

_mesh = plsc.VectorSubcoreMesh(core_axis_name="c", subcore_axis_name="s",
                               num_cores=NC, num_subcores=NS)


def _dense_nodes_body(x_ref, wl_ref, bl_ref, wr_ref, ul_ref, ur_ref):
    x = x_ref[...]
    ul_ref[...] = jnp.dot(x, wl_ref[...], preferred_element_type=jnp.float32) + bl_ref[...]
    ur_ref[...] = jnp.dot(x, wr_ref[...], preferred_element_type=jnp.float32)


def _dense_nodes(x, wl, bl, wr, nb):
    grid = N // nb
    return pl.pallas_call(
        _dense_nodes_body,
        grid=(grid,),
        in_specs=[
            pl.BlockSpec((nb, D), lambda i: (i, 0)),
            pl.BlockSpec((D, D), lambda i: (0, 0)),
            pl.BlockSpec((1, D), lambda i: (0, 0)),
            pl.BlockSpec((D, D), lambda i: (0, 0)),
        ],
        out_specs=[
            pl.BlockSpec((nb, D), lambda i: (i, 0)),
            pl.BlockSpec((nb, D), lambda i: (i, 0)),
        ],
        out_shape=[
            jax.ShapeDtypeStruct((N, D), jnp.float32),
            jax.ShapeDtypeStruct((N, D), jnp.float32),
        ],
    )(x, wl, bl.reshape(1, D), wr)


def _edge_stage_body(gl_ref, gr_ref, ea_ref, we_ref, att_ref, x_ref, wd_ref):
    gl = gl_ref[...]
    ue = jnp.dot(ea_ref[...], we_ref[...], preferred_element_type=jnp.float32)
    m = gl + gr_ref[...] + ue
    m = jnp.where(m >= 0.0, m, 0.2 * m)
    alpha = jnp.sum(m * att_ref[...], axis=1, keepdims=True)
    w = jnp.exp(alpha)
    x_ref[...] = w * gl
    wd_ref[...] = w


def _edge_stage(gl, gr, ea, we, att, eb):
    grid = E // eb
    return pl.pallas_call(
        _edge_stage_body,
        grid=(grid,),
        in_specs=[
            pl.BlockSpec((eb, D), lambda i: (i, 0)),
            pl.BlockSpec((eb, D), lambda i: (i, 0)),
            pl.BlockSpec((eb, ED), lambda i: (i, 0)),
            pl.BlockSpec((ED, D), lambda i: (0, 0)),
            pl.BlockSpec((1, D), lambda i: (0, 0)),
        ],
        out_specs=[
            pl.BlockSpec((eb, D), lambda i: (i, 0)),
            pl.BlockSpec((eb, 1), lambda i: (i, 0)),
        ],
        out_shape=[
            jax.ShapeDtypeStruct((E, D), jnp.float32),
            jax.ShapeDtypeStruct((E, 1), jnp.float32),
        ],
    )(gl, gr, ea, we, att.reshape(1, D))


def _node_update_body(elu, sn_ref, sd_ref, bias_ref, wl_ref, bl_ref, wr_ref,
                      ul_ref, ur_ref):
    num = sn_ref[0] + sn_ref[1]
    den = sd_ref[0] + sd_ref[1]
    h = num / (den + 1e-16) + bias_ref[...]
    if elu:
        h = jnp.where(h > 0.0, h, jnp.exp(h) - 1.0)
    ul_ref[...] = jnp.dot(h, wl_ref[...], preferred_element_type=jnp.float32) + bl_ref[...]
    ur_ref[...] = jnp.dot(h, wr_ref[...], preferred_element_type=jnp.float32)


def _node_update(sn, sd, bias, wl, bl, wr, nb, elu):
    grid = NP // nb
    return pl.pallas_call(
        functools.partial(_node_update_body, elu),
        grid=(grid,),
        in_specs=[
            pl.BlockSpec((NC, nb, D), lambda i: (0, i, 0)),
            pl.BlockSpec((NC, nb, 1), lambda i: (0, i, 0)),
            pl.BlockSpec((1, D), lambda i: (0, 0)),
            pl.BlockSpec((D, D), lambda i: (0, 0)),
            pl.BlockSpec((1, D), lambda i: (0, 0)),
            pl.BlockSpec((D, D), lambda i: (0, 0)),
        ],
        out_specs=[
            pl.BlockSpec((nb, D), lambda i: (i, 0)),
            pl.BlockSpec((nb, D), lambda i: (i, 0)),
        ],
        out_shape=[
            jax.ShapeDtypeStruct((NP, D), jnp.float32),
            jax.ShapeDtypeStruct((NP, D), jnp.float32),
        ],
    )(sn, sd, bias.reshape(1, D), wl, bl.reshape(1, D), wr)


def _pred_stage_body(gp_ref, gq_ref, wm2_ref, bm2_ref, out_ref):
    s = jnp.maximum(gp_ref[...] + gq_ref[...], 0.0)
    out_ref[...] = jnp.sum(s * wm2_ref[...], axis=1, keepdims=True) + bm2_ref[...]


def _pred_stage(gp, gq, wm2, bm2, eb):
    grid = E // eb
    return pl.pallas_call(
        _pred_stage_body,
        grid=(grid,),
        in_specs=[
            pl.BlockSpec((eb, D), lambda i: (i, 0)),
            pl.BlockSpec((eb, D), lambda i: (i, 0)),
            pl.BlockSpec((1, D), lambda i: (0, 0)),
            pl.BlockSpec((1, 1), lambda i: (0, 0)),
        ],
        out_specs=pl.BlockSpec((eb, 1), lambda i: (i, 0)),
        out_shape=jax.ShapeDtypeStruct((E, 1), jnp.float32),
    )(gp, gq, wm2.reshape(1, D), bm2.reshape(1, 1))


@functools.partial(
    pl.kernel,
    out_type=[
        jax.ShapeDtypeStruct((E, D), jnp.float32),
        jax.ShapeDtypeStruct((E, D), jnp.float32),
    ],
    mesh=_mesh,
    scratch_types=[
        pltpu.VMEM((NCHUNK, CH), jnp.int32),
        pltpu.VMEM((NCHUNK, CH), jnp.int32),
        pltpu.VMEM((CH, D), jnp.float32),
        pltpu.VMEM((CH, D), jnp.float32),
        pltpu.SemaphoreType.DMA,
        pltpu.SemaphoreType.DMA,
    ],
)
def _sc_gather2(taba_hbm, tabb_hbm, idxa_hbm, idxb_hbm, outa_hbm, outb_hbm,
                idxa_v, idxb_v, rowsa_v, rowsb_v, sema, semb):
    wid = lax.axis_index("s") * NC + lax.axis_index("c")
    pltpu.sync_copy(idxa_hbm.at[wid], idxa_v)
    pltpu.sync_copy(idxb_hbm.at[wid], idxb_v)
    base = wid * EW

    def body(j, carry):
        ca = pltpu.async_copy(taba_hbm.at[idxa_v.at[j]], rowsa_v, sema)
        cb = pltpu.async_copy(tabb_hbm.at[idxb_v.at[j]], rowsb_v, semb)
        ca.wait()
        pltpu.sync_copy(rowsa_v, outa_hbm.at[pl.ds(base + j * CH, CH)])
        cb.wait()
        pltpu.sync_copy(rowsb_v, outb_hbm.at[pl.ds(base + j * CH, CH)])
        return carry

    lax.fori_loop(0, NCHUNK, body, 0)


@functools.partial(
    pl.kernel,
    out_type=[
        jax.ShapeDtypeStruct((NC, NP, D), jnp.float32),
        jax.ShapeDtypeStruct((NC, NPR, 128), jnp.float32),
    ],
    mesh=_mesh,
    compiler_params=pltpu.CompilerParams(needs_layout_passes=False),
    scratch_types=[
        pltpu.VMEM_SHARED((NP, D), jnp.float32),
        pltpu.VMEM_SHARED((NPR, 128), jnp.float32),
        pltpu.VMEM((CH,), jnp.int32),
        pltpu.VMEM((CH, D), jnp.float32),
        pltpu.VMEM((CH,), jnp.float32),
        pltpu.VMEM((NPR, 128), jnp.float32),
        pltpu.VMEM((NPR,), jnp.int32),
    ],
)
def _sc_scatter(x_hbm, w_hbm, idx_hbm, zn_hbm, iota_hbm, sn_hbm, sd_hbm,
                accn_sh, accd_sh, idxc_v, ebuf_v, wbuf_v, den_v, iota_v):
    cid = lax.axis_index("c")
    sid = lax.axis_index("s")
    wid = sid * NC + cid
    stripe = sid * NSTRIPE
    base = wid * EW

    pltpu.sync_copy(zn_hbm, ebuf_v)
    pltpu.sync_copy(zn_hbm.at[pl.ds(0, NPR)], den_v)
    pltpu.sync_copy(iota_hbm, iota_v)

    def zbody(t, carry):
        pltpu.sync_copy(ebuf_v, accn_sh.at[pl.ds(stripe + t * SCH, SCH)])
        return carry

    lax.fori_loop(0, NSCH, zbody, 0)

    @pl.when(sid < NPR // DSTRIPE)
    def _():
        pltpu.sync_copy(ebuf_v.at[pl.ds(0, DSTRIPE)],
                        accd_sh.at[pl.ds(sid * DSTRIPE, DSTRIPE)])

    plsc.subcore_barrier()

    def body(j, carry):
        pltpu.sync_copy(idx_hbm.at[wid].at[j], idxc_v)
        pltpu.sync_copy(x_hbm.at[pl.ds(base + j * CH, CH)], ebuf_v)
        pltpu.sync_copy(ebuf_v, accn_sh.at[idxc_v], add=True)
        pltpu.sync_copy(w_hbm.at[wid].at[j], wbuf_v)
        for g in range(CH // 16):
            idx16 = idxc_v[pl.ds(g * 16, 16)]
            w16 = wbuf_v[pl.ds(g * 16, 16)]
            hi = lax.shift_right_logical(idx16, 7)
            lo = lax.bitwise_and(idx16, 127)
            plsc.addupdate_scatter(den_v, [hi, lo], w16)
        return carry

    lax.fori_loop(0, NCHUNK, body, 0)

    pltpu.sync_copy(den_v, accd_sh.at[iota_v], add=True)
    plsc.subcore_barrier()

    def obody(t, carry):
        pltpu.sync_copy(accn_sh.at[pl.ds(stripe + t * SCH, SCH)], ebuf_v)
        pltpu.sync_copy(ebuf_v, sn_hbm.at[cid].at[pl.ds(stripe + t * SCH, SCH)])
        return carry

    lax.fori_loop(0, NSCH, obody, 0)

    @pl.when(sid < NPR // DSTRIPE)
    def _():
        pltpu.sync_copy(accd_sh.at[pl.ds(sid * DSTRIPE, DSTRIPE)],
                        den_v.at[pl.ds(0, DSTRIPE)])
        pltpu.sync_copy(den_v.at[pl.ds(0, DSTRIPE)],
                        sd_hbm.at[cid].at[pl.ds(sid * DSTRIPE, DSTRIPE)])


def kernel(x, edge_index, edge_attr, W_l1, b_l1, W_r1, W_e1, att1, bias1,
           W_l2, b_l2, W_r2, W_e2, att2, bias2, Wm1, bm1, Wm2, bm2):
    src3 = edge_index[0].reshape(NW, NCHUNK, CH)
    dst3 = edge_index[1].reshape(NW, NCHUNK, CH)
    zn = jnp.zeros((SCH, D), jnp.float32)
    iota = jnp.arange(NPR, dtype=jnp.int32)

    ul1, ur1 = _dense_nodes(x, W_l1, b_l1, W_r1, nb=2000)
    gl1, gr1 = _sc_gather2(ul1, ur1, src3, dst3)
    x1, w1 = _edge_stage(gl1, gr1, edge_attr, W_e1, att1.reshape(D), eb=8000)
    sn1, sd1 = _sc_scatter(x1, w1.reshape(NW, NCHUNK, CH), dst3, zn, iota)

    ul2, ur2 = _node_update(sn1, sd1.reshape(NC, NP, 1), bias1, W_l2, b_l2,
                            W_r2, nb=2048, elu=True)
    gl2, gr2 = _sc_gather2(ul2, ur2, src3, dst3)
    x2, w2 = _edge_stage(gl2, gr2, edge_attr, W_e2, att2.reshape(D), eb=8000)
    sn2, sd2 = _sc_scatter(x2, w2.reshape(NW, NCHUNK, CH), dst3, zn, iota)

    p, q = _node_update(sn2, sd2.reshape(NC, NP, 1), bias2, Wm1[:D], bm1,
                        Wm1[D:], nb=2048, elu=False)
    gp, gq = _sc_gather2(p, q, src3, dst3)
    return _pred_stage(gp, gq, Wm2.reshape(D), bm2, eb=8000)


# speedup vs baseline: 6.3479x; 1.0428x over previous
import functools

import jax
import jax.numpy as jnp
from jax import lax
from jax.experimental import pallas as pl
from jax.experimental.pallas import tpu as pltpu
from jax.experimental.pallas import tpu_sc as plsc

N = 10000
E = 320000
D = 128
ED = 16

NC = 2
NS = 16
NW = NC * NS
EW = E // NW
CH = 80
NCHUNK = EW // CH
NP = 10240
NPR = NP // 128
NSTRIPE = NP // NS
SCH = 80
NSCH = NSTRIPE // SCH
DSTRIPE = 8

_mesh = plsc.VectorSubcoreMesh(core_axis_name="c", subcore_axis_name="s",
                               num_cores=NC, num_subcores=NS)


def _dense_nodes_body(x_ref, wl_ref, bl_ref, wr_ref, ul_ref, ur_ref):
    x = x_ref[...]
    ul_ref[...] = jnp.dot(x, wl_ref[...], preferred_element_type=jnp.float32) + bl_ref[...]
    ur_ref[...] = jnp.dot(x, wr_ref[...], preferred_element_type=jnp.float32)


def _dense_nodes(x, wl, bl, wr, nb):
    grid = N // nb
    return pl.pallas_call(
        _dense_nodes_body,
        grid=(grid,),
        in_specs=[
            pl.BlockSpec((nb, D), lambda i: (i, 0)),
            pl.BlockSpec((D, D), lambda i: (0, 0)),
            pl.BlockSpec((1, D), lambda i: (0, 0)),
            pl.BlockSpec((D, D), lambda i: (0, 0)),
        ],
        out_specs=[
            pl.BlockSpec((nb, D), lambda i: (i, 0)),
            pl.BlockSpec((nb, D), lambda i: (i, 0)),
        ],
        out_shape=[
            jax.ShapeDtypeStruct((N, D), jnp.float32),
            jax.ShapeDtypeStruct((N, D), jnp.float32),
        ],
    )(x, wl, bl.reshape(1, D), wr)


def _edge_stage_body(gl_ref, gr_ref, ea_ref, we_ref, att_ref, x_ref, wd_ref):
    gl = gl_ref[...]
    ue = jnp.dot(ea_ref[...], we_ref[...], preferred_element_type=jnp.float32)
    m = gl + gr_ref[...] + ue
    m = jnp.where(m >= 0.0, m, 0.2 * m)
    alpha = jnp.sum(m * att_ref[...], axis=1, keepdims=True)
    w = jnp.exp(alpha)
    x_ref[...] = w * gl
    wd_ref[...] = w


def _edge_stage(gl, gr, ea, we, att, eb):
    grid = E // eb
    return pl.pallas_call(
        _edge_stage_body,
        grid=(grid,),
        in_specs=[
            pl.BlockSpec((eb, D), lambda i: (i, 0)),
            pl.BlockSpec((eb, D), lambda i: (i, 0)),
            pl.BlockSpec((eb, ED), lambda i: (i, 0)),
            pl.BlockSpec((ED, D), lambda i: (0, 0)),
            pl.BlockSpec((1, D), lambda i: (0, 0)),
        ],
        out_specs=[
            pl.BlockSpec((eb, D), lambda i: (i, 0)),
            pl.BlockSpec((eb, 1), lambda i: (i, 0)),
        ],
        out_shape=[
            jax.ShapeDtypeStruct((E, D), jnp.float32),
            jax.ShapeDtypeStruct((E, 1), jnp.float32),
        ],
    )(gl, gr, ea, we, att.reshape(1, D))


def _node_update_body(elu, sn_ref, sd_ref, bias_ref, wl_ref, bl_ref, wr_ref,
                      ul_ref, ur_ref):
    num = sn_ref[0] + sn_ref[1]
    den = sd_ref[0] + sd_ref[1]
    h = num / (den + 1e-16) + bias_ref[...]
    if elu:
        h = jnp.where(h > 0.0, h, jnp.exp(h) - 1.0)
    ul_ref[...] = jnp.dot(h, wl_ref[...], preferred_element_type=jnp.float32) + bl_ref[...]
    ur_ref[...] = jnp.dot(h, wr_ref[...], preferred_element_type=jnp.float32)


def _node_update(sn, sd, bias, wl, bl, wr, nb, elu):
    grid = NP // nb
    return pl.pallas_call(
        functools.partial(_node_update_body, elu),
        grid=(grid,),
        in_specs=[
            pl.BlockSpec((NC, nb, D), lambda i: (0, i, 0)),
            pl.BlockSpec((NC, nb, 1), lambda i: (0, i, 0)),
            pl.BlockSpec((1, D), lambda i: (0, 0)),
            pl.BlockSpec((D, D), lambda i: (0, 0)),
            pl.BlockSpec((1, D), lambda i: (0, 0)),
            pl.BlockSpec((D, D), lambda i: (0, 0)),
        ],
        out_specs=[
            pl.BlockSpec((nb, D), lambda i: (i, 0)),
            pl.BlockSpec((nb, D), lambda i: (i, 0)),
        ],
        out_shape=[
            jax.ShapeDtypeStruct((NP, D), jnp.float32),
            jax.ShapeDtypeStruct((NP, D), jnp.float32),
        ],
    )(sn, sd, bias.reshape(1, D), wl, bl.reshape(1, D), wr)


def _pred_stage_body(gp_ref, gq_ref, wm2_ref, bm2_ref, out_ref):
    s = jnp.maximum(gp_ref[...] + gq_ref[...], 0.0)
    out_ref[...] = jnp.sum(s * wm2_ref[...], axis=1, keepdims=True) + bm2_ref[...]


def _pred_stage(gp, gq, wm2, bm2, eb):
    grid = E // eb
    return pl.pallas_call(
        _pred_stage_body,
        grid=(grid,),
        in_specs=[
            pl.BlockSpec((eb, D), lambda i: (i, 0)),
            pl.BlockSpec((eb, D), lambda i: (i, 0)),
            pl.BlockSpec((1, D), lambda i: (0, 0)),
            pl.BlockSpec((1, 1), lambda i: (0, 0)),
        ],
        out_specs=pl.BlockSpec((eb, 1), lambda i: (i, 0)),
        out_shape=jax.ShapeDtypeStruct((E, 1), jnp.float32),
    )(gp, gq, wm2.reshape(1, D), bm2.reshape(1, 1))


@functools.partial(
    pl.kernel,
    out_type=[
        jax.ShapeDtypeStruct((E, D), jnp.float32),
        jax.ShapeDtypeStruct((E, D), jnp.float32),
    ],
    mesh=_mesh,
    scratch_types=[
        pltpu.VMEM((NCHUNK, CH), jnp.int32),
        pltpu.VMEM((NCHUNK, CH), jnp.int32),
        pltpu.VMEM((CH, D), jnp.float32),
        pltpu.VMEM((CH, D), jnp.float32),
        pltpu.SemaphoreType.DMA,
        pltpu.SemaphoreType.DMA,
    ],
)
def _sc_gather2(taba_hbm, tabb_hbm, idxa_hbm, idxb_hbm, outa_hbm, outb_hbm,
                idxa_v, idxb_v, rowsa_v, rowsb_v, sema, semb):
    wid = lax.axis_index("s") * NC + lax.axis_index("c")
    pltpu.sync_copy(idxa_hbm.at[wid], idxa_v)
    pltpu.sync_copy(idxb_hbm.at[wid], idxb_v)
    base = wid * EW

    def body(j, carry):
        ca = pltpu.async_copy(taba_hbm.at[idxa_v.at[j]], rowsa_v, sema)
        cb = pltpu.async_copy(tabb_hbm.at[idxb_v.at[j]], rowsb_v, semb)
        ca.wait()
        pltpu.sync_copy(rowsa_v, outa_hbm.at[pl.ds(base + j * CH, CH)])
        cb.wait()
        pltpu.sync_copy(rowsb_v, outb_hbm.at[pl.ds(base + j * CH, CH)])
        return carry

    lax.fori_loop(0, NCHUNK, body, 0)


@functools.partial(
    pl.kernel,
    out_type=[
        jax.ShapeDtypeStruct((NC, NP, D), jnp.float32),
        jax.ShapeDtypeStruct((NC, NPR, 128), jnp.float32),
    ],
    mesh=_mesh,
    compiler_params=pltpu.CompilerParams(needs_layout_passes=False),
    scratch_types=[
        pltpu.VMEM_SHARED((NP, D), jnp.float32),
        pltpu.VMEM_SHARED((NPR, 128), jnp.float32),
        pltpu.VMEM((CH,), jnp.int32),
        pltpu.VMEM((CH, D), jnp.float32),
        pltpu.VMEM((CH,), jnp.float32),
        pltpu.VMEM((NPR, 128), jnp.float32),
        pltpu.VMEM((NPR,), jnp.int32),
        pltpu.SemaphoreType.DMA,
    ],
)
def _sc_scatter(x_hbm, w_hbm, idx_hbm, zn_hbm, iota_hbm, sn_hbm, sd_hbm,
                accn_sh, accd_sh, idxc_v, ebuf_v, wbuf_v, den_v, iota_v, semn):
    cid = lax.axis_index("c")
    sid = lax.axis_index("s")
    wid = sid * NC + cid
    stripe = sid * NSTRIPE
    base = wid * EW

    pltpu.sync_copy(zn_hbm, ebuf_v)
    pltpu.sync_copy(zn_hbm.at[pl.ds(0, NPR)], den_v)
    pltpu.sync_copy(iota_hbm, iota_v)

    def zbody(t, carry):
        pltpu.sync_copy(ebuf_v, accn_sh.at[pl.ds(stripe + t * SCH, SCH)])
        return carry

    lax.fori_loop(0, NSCH, zbody, 0)

    @pl.when(sid < NPR // DSTRIPE)
    def _():
        pltpu.sync_copy(ebuf_v.at[pl.ds(0, DSTRIPE)],
                        accd_sh.at[pl.ds(sid * DSTRIPE, DSTRIPE)])

    plsc.subcore_barrier()

    def body(j, carry):
        pltpu.sync_copy(idx_hbm.at[wid].at[j], idxc_v)
        pltpu.sync_copy(x_hbm.at[pl.ds(base + j * CH, CH)], ebuf_v)
        cn = pltpu.async_copy(ebuf_v, accn_sh.at[idxc_v], semn, add=True)
        pltpu.sync_copy(w_hbm.at[wid].at[j], wbuf_v)
        for g in range(CH // 16):
            idx16 = idxc_v[pl.ds(g * 16, 16)]
            w16 = wbuf_v[pl.ds(g * 16, 16)]
            hi = lax.shift_right_logical(idx16, 7)
            lo = lax.bitwise_and(idx16, 127)
            plsc.addupdate_scatter(den_v, [hi, lo], w16)
        cn.wait()
        return carry

    lax.fori_loop(0, NCHUNK, body, 0)

    pltpu.sync_copy(den_v, accd_sh.at[iota_v], add=True)
    plsc.subcore_barrier()

    def obody(t, carry):
        pltpu.sync_copy(accn_sh.at[pl.ds(stripe + t * SCH, SCH)], ebuf_v)
        pltpu.sync_copy(ebuf_v, sn_hbm.at[cid].at[pl.ds(stripe + t * SCH, SCH)])
        return carry

    lax.fori_loop(0, NSCH, obody, 0)

    @pl.when(sid < NPR // DSTRIPE)
    def _():
        pltpu.sync_copy(accd_sh.at[pl.ds(sid * DSTRIPE, DSTRIPE)],
                        den_v.at[pl.ds(0, DSTRIPE)])
        pltpu.sync_copy(den_v.at[pl.ds(0, DSTRIPE)],
                        sd_hbm.at[cid].at[pl.ds(sid * DSTRIPE, DSTRIPE)])


def kernel(x, edge_index, edge_attr, W_l1, b_l1, W_r1, W_e1, att1, bias1,
           W_l2, b_l2, W_r2, W_e2, att2, bias2, Wm1, bm1, Wm2, bm2):
    src3 = edge_index[0].reshape(NW, NCHUNK, CH)
    dst3 = edge_index[1].reshape(NW, NCHUNK, CH)
    zn = jnp.zeros((SCH, D), jnp.float32)
    iota = jnp.arange(NPR, dtype=jnp.int32)

    ul1, ur1 = _dense_nodes(x, W_l1, b_l1, W_r1, nb=2000)
    gl1, gr1 = _sc_gather2(ul1, ur1, src3, dst3)
    x1, w1 = _edge_stage(gl1, gr1, edge_attr, W_e1, att1.reshape(D), eb=8000)
    sn1, sd1 = _sc_scatter(x1, w1.reshape(NW, NCHUNK, CH), dst3, zn, iota)

    ul2, ur2 = _node_update(sn1, sd1.reshape(NC, NP, 1), bias1, W_l2, b_l2,
                            W_r2, nb=2048, elu=True)
    gl2, gr2 = _sc_gather2(ul2, ur2, src3, dst3)
    x2, w2 = _edge_stage(gl2, gr2, edge_attr, W_e2, att2.reshape(D), eb=8000)
    sn2, sd2 = _sc_scatter(x2, w2.reshape(NW, NCHUNK, CH), dst3, zn, iota)

    p, q = _node_update(sn2, sd2.reshape(NC, NP, 1), bias2, Wm1[:D], bm1,
                        Wm1[D:], nb=2048, elu=False)
    gp, gq = _sc_gather2(p, q, src3, dst3)
    return _pred_stage(gp, gq, Wm2.reshape(D), bm2, eb=8000)


# speedup vs baseline: 6.3834x; 1.0056x over previous
import functools

import jax
import jax.numpy as jnp
from jax import lax
from jax.experimental import pallas as pl
from jax.experimental.pallas import tpu as pltpu
from jax.experimental.pallas import tpu_sc as plsc

N = 10000
E = 320000
D = 128
ED = 16

NC = 2
NS = 16
NW = NC * NS
EW = E // NW
CH = 80
NCHUNK = EW // CH
NP = 10240
NPR = NP // 128
NSTRIPE = NP // NS
SCH = 80
NSCH = NSTRIPE // SCH
DSTRIPE = 8

_mesh = plsc.VectorSubcoreMesh(core_axis_name="c", subcore_axis_name="s",
                               num_cores=NC, num_subcores=NS)


def _dense_nodes_body(x_ref, wl_ref, bl_ref, wr_ref, ul_ref, ur_ref):
    x = x_ref[...]
    ul_ref[...] = jnp.dot(x, wl_ref[...], preferred_element_type=jnp.float32) + bl_ref[...]
    ur_ref[...] = jnp.dot(x, wr_ref[...], preferred_element_type=jnp.float32)


def _dense_nodes(x, wl, bl, wr, nb):
    grid = N // nb
    return pl.pallas_call(
        _dense_nodes_body,
        grid=(grid,),
        in_specs=[
            pl.BlockSpec((nb, D), lambda i: (i, 0)),
            pl.BlockSpec((D, D), lambda i: (0, 0)),
            pl.BlockSpec((1, D), lambda i: (0, 0)),
            pl.BlockSpec((D, D), lambda i: (0, 0)),
        ],
        out_specs=[
            pl.BlockSpec((nb, D), lambda i: (i, 0)),
            pl.BlockSpec((nb, D), lambda i: (i, 0)),
        ],
        out_shape=[
            jax.ShapeDtypeStruct((N, D), jnp.float32),
            jax.ShapeDtypeStruct((N, D), jnp.float32),
        ],
    )(x, wl, bl.reshape(1, D), wr)


def _edge_stage_body(gl_ref, gr_ref, ea_ref, we_ref, att_ref, x_ref, wd_ref):
    gl = gl_ref[...]
    ue = jnp.dot(ea_ref[...], we_ref[...], preferred_element_type=jnp.float32)
    m = gl + gr_ref[...] + ue
    m = jnp.where(m >= 0.0, m, 0.2 * m)
    alpha = jnp.sum(m * att_ref[...], axis=1, keepdims=True)
    w = jnp.exp(alpha)
    x_ref[...] = w * gl
    wd_ref[...] = w


def _edge_stage(gl, gr, ea, we, att, eb):
    grid = E // eb
    return pl.pallas_call(
        _edge_stage_body,
        grid=(grid,),
        in_specs=[
            pl.BlockSpec((eb, D), lambda i: (i, 0)),
            pl.BlockSpec((eb, D), lambda i: (i, 0)),
            pl.BlockSpec((eb, ED), lambda i: (i, 0)),
            pl.BlockSpec((ED, D), lambda i: (0, 0)),
            pl.BlockSpec((1, D), lambda i: (0, 0)),
        ],
        out_specs=[
            pl.BlockSpec((eb, D), lambda i: (i, 0)),
            pl.BlockSpec((eb, 1), lambda i: (i, 0)),
        ],
        out_shape=[
            jax.ShapeDtypeStruct((E, D), jnp.float32),
            jax.ShapeDtypeStruct((E, 1), jnp.float32),
        ],
    )(gl, gr, ea, we, att.reshape(1, D))


def _node_update_body(elu, sn_ref, sd_ref, bias_ref, wl_ref, bl_ref, wr_ref,
                      ul_ref, ur_ref):
    num = sn_ref[0] + sn_ref[1]
    den = sd_ref[0] + sd_ref[1]
    h = num / (den + 1e-16) + bias_ref[...]
    if elu:
        h = jnp.where(h > 0.0, h, jnp.exp(h) - 1.0)
    ul_ref[...] = jnp.dot(h, wl_ref[...], preferred_element_type=jnp.float32) + bl_ref[...]
    ur_ref[...] = jnp.dot(h, wr_ref[...], preferred_element_type=jnp.float32)


def _node_update(sn, sd, bias, wl, bl, wr, nb, elu):
    grid = NP // nb
    return pl.pallas_call(
        functools.partial(_node_update_body, elu),
        grid=(grid,),
        in_specs=[
            pl.BlockSpec((NC, nb, D), lambda i: (0, i, 0)),
            pl.BlockSpec((NC, nb, 1), lambda i: (0, i, 0)),
            pl.BlockSpec((1, D), lambda i: (0, 0)),
            pl.BlockSpec((D, D), lambda i: (0, 0)),
            pl.BlockSpec((1, D), lambda i: (0, 0)),
            pl.BlockSpec((D, D), lambda i: (0, 0)),
        ],
        out_specs=[
            pl.BlockSpec((nb, D), lambda i: (i, 0)),
            pl.BlockSpec((nb, D), lambda i: (i, 0)),
        ],
        out_shape=[
            jax.ShapeDtypeStruct((NP, D), jnp.float32),
            jax.ShapeDtypeStruct((NP, D), jnp.float32),
        ],
    )(sn, sd, bias.reshape(1, D), wl, bl.reshape(1, D), wr)


def _pred_stage_body(gp_ref, gq_ref, wm2_ref, bm2_ref, out_ref):
    s = jnp.maximum(gp_ref[...] + gq_ref[...], 0.0)
    out_ref[...] = jnp.sum(s * wm2_ref[...], axis=1, keepdims=True) + bm2_ref[...]


def _pred_stage(gp, gq, wm2, bm2, eb):
    grid = E // eb
    return pl.pallas_call(
        _pred_stage_body,
        grid=(grid,),
        in_specs=[
            pl.BlockSpec((eb, D), lambda i: (i, 0)),
            pl.BlockSpec((eb, D), lambda i: (i, 0)),
            pl.BlockSpec((1, D), lambda i: (0, 0)),
            pl.BlockSpec((1, 1), lambda i: (0, 0)),
        ],
        out_specs=pl.BlockSpec((eb, 1), lambda i: (i, 0)),
        out_shape=jax.ShapeDtypeStruct((E, 1), jnp.float32),
    )(gp, gq, wm2.reshape(1, D), bm2.reshape(1, 1))


@functools.partial(
    pl.kernel,
    out_type=[
        jax.ShapeDtypeStruct((E, D), jnp.float32),
        jax.ShapeDtypeStruct((E, D), jnp.float32),
    ],
    mesh=_mesh,
    scratch_types=[
        pltpu.VMEM((NCHUNK, CH), jnp.int32),
        pltpu.VMEM((NCHUNK, CH), jnp.int32),
        pltpu.VMEM((CH, D), jnp.float32),
        pltpu.VMEM((CH, D), jnp.float32),
        pltpu.SemaphoreType.DMA,
        pltpu.SemaphoreType.DMA,
        pltpu.SemaphoreType.DMA,
        pltpu.SemaphoreType.DMA,
    ],
)
def _sc_gather2(taba_hbm, tabb_hbm, idxa_hbm, idxb_hbm, outa_hbm, outb_hbm,
                idxa_v, idxb_v, rowsa_v, rowsb_v, sema, semb, semwa, semwb):
    wid = lax.axis_index("s") * NC + lax.axis_index("c")
    pltpu.sync_copy(idxa_hbm.at[wid], idxa_v)
    pltpu.sync_copy(idxb_hbm.at[wid], idxb_v)
    base = wid * EW

    def body(j, carry):
        ca = pltpu.async_copy(taba_hbm.at[idxa_v.at[j]], rowsa_v, sema)
        cb = pltpu.async_copy(tabb_hbm.at[idxb_v.at[j]], rowsb_v, semb)
        ca.wait()
        wa = pltpu.async_copy(rowsa_v, outa_hbm.at[pl.ds(base + j * CH, CH)],
                              semwa)
        cb.wait()
        wb = pltpu.async_copy(rowsb_v, outb_hbm.at[pl.ds(base + j * CH, CH)],
                              semwb)
        wa.wait()
        wb.wait()
        return carry

    lax.fori_loop(0, NCHUNK, body, 0)


@functools.partial(
    pl.kernel,
    out_type=[
        jax.ShapeDtypeStruct((NC, NP, D), jnp.float32),
        jax.ShapeDtypeStruct((NC, NPR, 128), jnp.float32),
    ],
    mesh=_mesh,
    compiler_params=pltpu.CompilerParams(needs_layout_passes=False),
    scratch_types=[
        pltpu.VMEM_SHARED((NP, D), jnp.float32),
        pltpu.VMEM_SHARED((NPR, 128), jnp.float32),
        pltpu.VMEM((CH,), jnp.int32),
        pltpu.VMEM((CH, D), jnp.float32),
        pltpu.VMEM((CH,), jnp.float32),
        pltpu.VMEM((NPR, 128), jnp.float32),
        pltpu.VMEM((NPR,), jnp.int32),
        pltpu.SemaphoreType.DMA,
    ],
)
def _sc_scatter(x_hbm, w_hbm, idx_hbm, zn_hbm, iota_hbm, sn_hbm, sd_hbm,
                accn_sh, accd_sh, idxc_v, ebuf_v, wbuf_v, den_v, iota_v, semn):
    cid = lax.axis_index("c")
    sid = lax.axis_index("s")
    wid = sid * NC + cid
    stripe = sid * NSTRIPE
    base = wid * EW

    pltpu.sync_copy(zn_hbm, ebuf_v)
    pltpu.sync_copy(zn_hbm.at[pl.ds(0, NPR)], den_v)
    pltpu.sync_copy(iota_hbm, iota_v)

    def zbody(t, carry):
        pltpu.sync_copy(ebuf_v, accn_sh.at[pl.ds(stripe + t * SCH, SCH)])
        return carry

    lax.fori_loop(0, NSCH, zbody, 0)

    @pl.when(sid < NPR // DSTRIPE)
    def _():
        pltpu.sync_copy(ebuf_v.at[pl.ds(0, DSTRIPE)],
                        accd_sh.at[pl.ds(sid * DSTRIPE, DSTRIPE)])

    plsc.subcore_barrier()

    def body(j, carry):
        pltpu.sync_copy(idx_hbm.at[wid].at[j], idxc_v)
        pltpu.sync_copy(x_hbm.at[pl.ds(base + j * CH, CH)], ebuf_v)
        cn = pltpu.async_copy(ebuf_v, accn_sh.at[idxc_v], semn, add=True)
        pltpu.sync_copy(w_hbm.at[wid].at[j], wbuf_v)
        for g in range(CH // 16):
            idx16 = idxc_v[pl.ds(g * 16, 16)]
            w16 = wbuf_v[pl.ds(g * 16, 16)]
            hi = lax.shift_right_logical(idx16, 7)
            lo = lax.bitwise_and(idx16, 127)
            plsc.addupdate_scatter(den_v, [hi, lo], w16)
        cn.wait()
        return carry

    lax.fori_loop(0, NCHUNK, body, 0)

    pltpu.sync_copy(den_v, accd_sh.at[iota_v], add=True)
    plsc.subcore_barrier()

    def obody(t, carry):
        pltpu.sync_copy(accn_sh.at[pl.ds(stripe + t * SCH, SCH)], ebuf_v)
        pltpu.sync_copy(ebuf_v, sn_hbm.at[cid].at[pl.ds(stripe + t * SCH, SCH)])
        return carry

    lax.fori_loop(0, NSCH, obody, 0)

    @pl.when(sid < NPR // DSTRIPE)
    def _():
        pltpu.sync_copy(accd_sh.at[pl.ds(sid * DSTRIPE, DSTRIPE)],
                        den_v.at[pl.ds(0, DSTRIPE)])
        pltpu.sync_copy(den_v.at[pl.ds(0, DSTRIPE)],
                        sd_hbm.at[cid].at[pl.ds(sid * DSTRIPE, DSTRIPE)])


def kernel(x, edge_index, edge_attr, W_l1, b_l1, W_r1, W_e1, att1, bias1,
           W_l2, b_l2, W_r2, W_e2, att2, bias2, Wm1, bm1, Wm2, bm2):
    src3 = edge_index[0].reshape(NW, NCHUNK, CH)
    dst3 = edge_index[1].reshape(NW, NCHUNK, CH)
    zn = jnp.zeros((SCH, D), jnp.float32)
    iota = jnp.arange(NPR, dtype=jnp.int32)

    ul1, ur1 = _dense_nodes(x, W_l1, b_l1, W_r1, nb=2000)
    gl1, gr1 = _sc_gather2(ul1, ur1, src3, dst3)
    x1, w1 = _edge_stage(gl1, gr1, edge_attr, W_e1, att1.reshape(D), eb=8000)
    sn1, sd1 = _sc_scatter(x1, w1.reshape(NW, NCHUNK, CH), dst3, zn, iota)

    ul2, ur2 = _node_update(sn1, sd1.reshape(NC, NP, 1), bias1, W_l2, b_l2,
                            W_r2, nb=2048, elu=True)
    gl2, gr2 = _sc_gather2(ul2, ur2, src3, dst3)
    x2, w2 = _edge_stage(gl2, gr2, edge_attr, W_e2, att2.reshape(D), eb=8000)
    sn2, sd2 = _sc_scatter(x2, w2.reshape(NW, NCHUNK, CH), dst3, zn, iota)

    p, q = _node_update(sn2, sd2.reshape(NC, NP, 1), bias2, Wm1[:D], bm1,
                        Wm1[D:], nb=2048, elu=False)
    gp, gq = _sc_gather2(p, q, src3, dst3)
    return _pred_stage(gp, gq, Wm2.reshape(D), bm2, eb=8000)


# speedup vs baseline: 6.6956x; 1.0489x over previous
import functools

import jax
import jax.numpy as jnp
from jax import lax
from jax.experimental import pallas as pl
from jax.experimental.pallas import tpu as pltpu
from jax.experimental.pallas import tpu_sc as plsc

N = 10000
E = 320000
D = 128
ED = 16

NC = 2
NS = 16
NW = NC * NS
EW = E // NW
CH = 80
NCHUNK = EW // CH
NP = 10240
NPR = NP // 128
NSTRIPE = NP // NS
SCH = 80
NSCH = NSTRIPE // SCH
DSTRIPE = 8

_mesh = plsc.VectorSubcoreMesh(core_axis_name="c", subcore_axis_name="s",
                               num_cores=NC, num_subcores=NS)


def _dense_nodes_body(x_ref, wl_ref, bl_ref, wr_ref, ul_ref, ur_ref):
    x = x_ref[...]
    ul_ref[...] = jnp.dot(x, wl_ref[...], preferred_element_type=jnp.float32) + bl_ref[...]
    ur_ref[...] = jnp.dot(x, wr_ref[...], preferred_element_type=jnp.float32)


def _dense_nodes(x, wl, bl, wr, nb):
    grid = N // nb
    return pl.pallas_call(
        _dense_nodes_body,
        grid=(grid,),
        in_specs=[
            pl.BlockSpec((nb, D), lambda i: (i, 0)),
            pl.BlockSpec((D, D), lambda i: (0, 0)),
            pl.BlockSpec((1, D), lambda i: (0, 0)),
            pl.BlockSpec((D, D), lambda i: (0, 0)),
        ],
        out_specs=[
            pl.BlockSpec((nb, D), lambda i: (i, 0)),
            pl.BlockSpec((nb, D), lambda i: (i, 0)),
        ],
        out_shape=[
            jax.ShapeDtypeStruct((N, D), jnp.float32),
            jax.ShapeDtypeStruct((N, D), jnp.float32),
        ],
    )(x, wl, bl.reshape(1, D), wr)


def _edge_stage_body(gl_ref, gr_ref, ea_ref, we_ref, att_ref, x_ref, wd_ref):
    gl = gl_ref[...]
    ue = jnp.dot(ea_ref[...], we_ref[...], preferred_element_type=jnp.float32)
    m = gl + gr_ref[...] + ue
    m = jnp.where(m >= 0.0, m, 0.2 * m)
    alpha = jnp.sum(m * att_ref[...], axis=1, keepdims=True)
    w = jnp.exp(alpha)
    x_ref[...] = w * gl
    wd_ref[...] = w


def _edge_stage(gl, gr, ea, we, att, eb):
    grid = E // eb
    return pl.pallas_call(
        _edge_stage_body,
        grid=(grid,),
        in_specs=[
            pl.BlockSpec((eb, D), lambda i: (i, 0)),
            pl.BlockSpec((eb, D), lambda i: (i, 0)),
            pl.BlockSpec((eb, ED), lambda i: (i, 0)),
            pl.BlockSpec((ED, D), lambda i: (0, 0)),
            pl.BlockSpec((1, D), lambda i: (0, 0)),
        ],
        out_specs=[
            pl.BlockSpec((eb, D), lambda i: (i, 0)),
            pl.BlockSpec((eb, 1), lambda i: (i, 0)),
        ],
        out_shape=[
            jax.ShapeDtypeStruct((E, D), jnp.float32),
            jax.ShapeDtypeStruct((E, 1), jnp.float32),
        ],
    )(gl, gr, ea, we, att.reshape(1, D))


def _node_update_body(elu, sn_ref, sd_ref, bias_ref, wl_ref, bl_ref, wr_ref,
                      ul_ref, ur_ref):
    num = sn_ref[0] + sn_ref[1]
    den = sd_ref[0] + sd_ref[1]
    h = num / (den + 1e-16) + bias_ref[...]
    if elu:
        h = jnp.where(h > 0.0, h, jnp.exp(h) - 1.0)
    ul_ref[...] = jnp.dot(h, wl_ref[...], preferred_element_type=jnp.float32) + bl_ref[...]
    ur_ref[...] = jnp.dot(h, wr_ref[...], preferred_element_type=jnp.float32)


def _node_update(sn, sd, bias, wl, bl, wr, nb, elu):
    grid = NP // nb
    return pl.pallas_call(
        functools.partial(_node_update_body, elu),
        grid=(grid,),
        in_specs=[
            pl.BlockSpec((NC, nb, D), lambda i: (0, i, 0)),
            pl.BlockSpec((NC, nb, 1), lambda i: (0, i, 0)),
            pl.BlockSpec((1, D), lambda i: (0, 0)),
            pl.BlockSpec((D, D), lambda i: (0, 0)),
            pl.BlockSpec((1, D), lambda i: (0, 0)),
            pl.BlockSpec((D, D), lambda i: (0, 0)),
        ],
        out_specs=[
            pl.BlockSpec((nb, D), lambda i: (i, 0)),
            pl.BlockSpec((nb, D), lambda i: (i, 0)),
        ],
        out_shape=[
            jax.ShapeDtypeStruct((NP, D), jnp.float32),
            jax.ShapeDtypeStruct((NP, D), jnp.float32),
        ],
    )(sn, sd, bias.reshape(1, D), wl, bl.reshape(1, D), wr)


def _pred_stage_body(gp_ref, gq_ref, wm2_ref, bm2_ref, out_ref):
    s = jnp.maximum(gp_ref[...] + gq_ref[...], 0.0)
    out_ref[...] = jnp.sum(s * wm2_ref[...], axis=1, keepdims=True) + bm2_ref[...]


def _pred_stage(gp, gq, wm2, bm2, eb):
    grid = E // eb
    return pl.pallas_call(
        _pred_stage_body,
        grid=(grid,),
        in_specs=[
            pl.BlockSpec((eb, D), lambda i: (i, 0)),
            pl.BlockSpec((eb, D), lambda i: (i, 0)),
            pl.BlockSpec((1, D), lambda i: (0, 0)),
            pl.BlockSpec((1, 1), lambda i: (0, 0)),
        ],
        out_specs=pl.BlockSpec((eb, 1), lambda i: (i, 0)),
        out_shape=jax.ShapeDtypeStruct((E, 1), jnp.float32),
    )(gp, gq, wm2.reshape(1, D), bm2.reshape(1, 1))


@functools.partial(
    pl.kernel,
    out_type=[
        jax.ShapeDtypeStruct((E, D), jnp.float32),
        jax.ShapeDtypeStruct((E, D), jnp.float32),
    ],
    mesh=_mesh,
    scratch_types=[
        pltpu.VMEM((NCHUNK, CH), jnp.int32),
        pltpu.VMEM((NCHUNK, CH), jnp.int32),
        pltpu.VMEM((CH, D), jnp.float32),
        pltpu.VMEM((CH, D), jnp.float32),
        pltpu.VMEM((CH, D), jnp.float32),
        pltpu.VMEM((CH, D), jnp.float32),
        pltpu.SemaphoreType.DMA,
        pltpu.SemaphoreType.DMA,
        pltpu.SemaphoreType.DMA,
        pltpu.SemaphoreType.DMA,
        pltpu.SemaphoreType.DMA,
        pltpu.SemaphoreType.DMA,
    ],
)
def _sc_gather2(taba_hbm, tabb_hbm, idxa_hbm, idxb_hbm, outa_hbm, outb_hbm,
                idxa_v, idxb_v, rowsa0_v, rowsb0_v, rowsa1_v, rowsb1_v,
                sema0, semb0, sema1, semb1, semwa, semwb):
    wid = lax.axis_index("s") * NC + lax.axis_index("c")
    pltpu.sync_copy(idxa_hbm.at[wid], idxa_v)
    pltpu.sync_copy(idxb_hbm.at[wid], idxb_v)
    base = wid * EW

    def body(t, carry):
        j0 = 2 * t
        j1 = j0 + 1
        ca0 = pltpu.async_copy(taba_hbm.at[idxa_v.at[j0]], rowsa0_v, sema0)
        cb0 = pltpu.async_copy(tabb_hbm.at[idxb_v.at[j0]], rowsb0_v, semb0)

        @pl.when(j1 < NCHUNK)
        def _():
            ca1 = pltpu.async_copy(taba_hbm.at[idxa_v.at[j1]], rowsa1_v, sema1)
            cb1 = pltpu.async_copy(tabb_hbm.at[idxb_v.at[j1]], rowsb1_v, semb1)
            ca0.wait()
            wa0 = pltpu.async_copy(
                rowsa0_v, outa_hbm.at[pl.ds(base + j0 * CH, CH)], semwa)
            cb0.wait()
            wb0 = pltpu.async_copy(
                rowsb0_v, outb_hbm.at[pl.ds(base + j0 * CH, CH)], semwb)
            ca1.wait()
            wa0.wait()
            wa1 = pltpu.async_copy(
                rowsa1_v, outa_hbm.at[pl.ds(base + j1 * CH, CH)], semwa)
            cb1.wait()
            wb0.wait()
            wb1 = pltpu.async_copy(
                rowsb1_v, outb_hbm.at[pl.ds(base + j1 * CH, CH)], semwb)
            wa1.wait()
            wb1.wait()

        @pl.when(j1 >= NCHUNK)
        def _():
            ca0.wait()
            wa0 = pltpu.async_copy(
                rowsa0_v, outa_hbm.at[pl.ds(base + j0 * CH, CH)], semwa)
            cb0.wait()
            wb0 = pltpu.async_copy(
                rowsb0_v, outb_hbm.at[pl.ds(base + j0 * CH, CH)], semwb)
            wa0.wait()
            wb0.wait()

        return carry

    lax.fori_loop(0, (NCHUNK + 1) // 2, body, 0)


@functools.partial(
    pl.kernel,
    out_type=[
        jax.ShapeDtypeStruct((NC, NP, D), jnp.float32),
        jax.ShapeDtypeStruct((NC, NPR, 128), jnp.float32),
    ],
    mesh=_mesh,
    compiler_params=pltpu.CompilerParams(needs_layout_passes=False),
    scratch_types=[
        pltpu.VMEM_SHARED((NP, D), jnp.float32),
        pltpu.VMEM_SHARED((NPR, 128), jnp.float32),
        pltpu.VMEM((CH,), jnp.int32),
        pltpu.VMEM((CH, D), jnp.float32),
        pltpu.VMEM((CH,), jnp.float32),
        pltpu.VMEM((NPR, 128), jnp.float32),
        pltpu.VMEM((NPR,), jnp.int32),
        pltpu.SemaphoreType.DMA,
    ],
)
def _sc_scatter(x_hbm, w_hbm, idx_hbm, zn_hbm, iota_hbm, sn_hbm, sd_hbm,
                accn_sh, accd_sh, idxc_v, ebuf_v, wbuf_v, den_v, iota_v, semn):
    cid = lax.axis_index("c")
    sid = lax.axis_index("s")
    wid = sid * NC + cid
    stripe = sid * NSTRIPE
    base = wid * EW

    pltpu.sync_copy(zn_hbm, ebuf_v)
    pltpu.sync_copy(zn_hbm.at[pl.ds(0, NPR)], den_v)
    pltpu.sync_copy(iota_hbm, iota_v)

    def zbody(t, carry):
        pltpu.sync_copy(ebuf_v, accn_sh.at[pl.ds(stripe + t * SCH, SCH)])
        return carry

    lax.fori_loop(0, NSCH, zbody, 0)

    @pl.when(sid < NPR // DSTRIPE)
    def _():
        pltpu.sync_copy(ebuf_v.at[pl.ds(0, DSTRIPE)],
                        accd_sh.at[pl.ds(sid * DSTRIPE, DSTRIPE)])

    plsc.subcore_barrier()

    def body(j, carry):
        pltpu.sync_copy(idx_hbm.at[wid].at[j], idxc_v)
        pltpu.sync_copy(x_hbm.at[pl.ds(base + j * CH, CH)], ebuf_v)
        cn = pltpu.async_copy(ebuf_v, accn_sh.at[idxc_v], semn, add=True)
        pltpu.sync_copy(w_hbm.at[wid].at[j], wbuf_v)
        for g in range(CH // 16):
            idx16 = idxc_v[pl.ds(g * 16, 16)]
            w16 = wbuf_v[pl.ds(g * 16, 16)]
            hi = lax.shift_right_logical(idx16, 7)
            lo = lax.bitwise_and(idx16, 127)
            plsc.addupdate_scatter(den_v, [hi, lo], w16)
        cn.wait()
        return carry

    lax.fori_loop(0, NCHUNK, body, 0)

    pltpu.sync_copy(den_v, accd_sh.at[iota_v], add=True)
    plsc.subcore_barrier()

    def obody(t, carry):
        pltpu.sync_copy(accn_sh.at[pl.ds(stripe + t * SCH, SCH)], ebuf_v)
        pltpu.sync_copy(ebuf_v, sn_hbm.at[cid].at[pl.ds(stripe + t * SCH, SCH)])
        return carry

    lax.fori_loop(0, NSCH, obody, 0)

    @pl.when(sid < NPR // DSTRIPE)
    def _():
        pltpu.sync_copy(accd_sh.at[pl.ds(sid * DSTRIPE, DSTRIPE)],
                        den_v.at[pl.ds(0, DSTRIPE)])
        pltpu.sync_copy(den_v.at[pl.ds(0, DSTRIPE)],
                        sd_hbm.at[cid].at[pl.ds(sid * DSTRIPE, DSTRIPE)])


def kernel(x, edge_index, edge_attr, W_l1, b_l1, W_r1, W_e1, att1, bias1,
           W_l2, b_l2, W_r2, W_e2, att2, bias2, Wm1, bm1, Wm2, bm2):
    src3 = edge_index[0].reshape(NW, NCHUNK, CH)
    dst3 = edge_index[1].reshape(NW, NCHUNK, CH)
    zn = jnp.zeros((SCH, D), jnp.float32)
    iota = jnp.arange(NPR, dtype=jnp.int32)

    ul1, ur1 = _dense_nodes(x, W_l1, b_l1, W_r1, nb=2000)
    gl1, gr1 = _sc_gather2(ul1, ur1, src3, dst3)
    x1, w1 = _edge_stage(gl1, gr1, edge_attr, W_e1, att1.reshape(D), eb=8000)
    sn1, sd1 = _sc_scatter(x1, w1.reshape(NW, NCHUNK, CH), dst3, zn, iota)

    ul2, ur2 = _node_update(sn1, sd1.reshape(NC, NP, 1), bias1, W_l2, b_l2,
                            W_r2, nb=2048, elu=True)
    gl2, gr2 = _sc_gather2(ul2, ur2, src3, dst3)
    x2, w2 = _edge_stage(gl2, gr2, edge_attr, W_e2, att2.reshape(D), eb=8000)
    sn2, sd2 = _sc_scatter(x2, w2.reshape(NW, NCHUNK, CH), dst3, zn, iota)

    p, q = _node_update(sn2, sd2.reshape(NC, NP, 1), bias2, Wm1[:D], bm1,
                        Wm1[D:], nb=2048, elu=False)
    gp, gq = _sc_gather2(p, q, src3, dst3)
    return _pred_stage(gp, gq, Wm2.reshape(D), bm2, eb=8000)
